# Initial kernel scaffold; baseline (speedup 1.0000x reference)
import jax
import jax.numpy as jnp
from jax.experimental import pallas as pl


def kernel(x, emb_table, W_up, b_up):
    raise NotImplementedError("write your pallas kernel here")


# speedup vs baseline: 3.5352x; 3.5352x over previous
import functools

import jax
import jax.numpy as jnp
from jax import lax
from jax.experimental import pallas as pl
from jax.experimental.pallas import tpu as pltpu
from jax.experimental.pallas import tpu_sc as plsc

HID = 128
OUT = 512

_NC = 2
_NS = 16
_NW = _NC * _NS

_K = 128
_NB = 4


def _sc_gather(x2d, emb_table, n_rows):
    n_chunks = x2d.shape[0]
    chunks_per_w = n_chunks // _NW
    rows_per_w = chunks_per_w * _K
    n_groups = chunks_per_w // _NB

    mesh = plsc.VectorSubcoreMesh(core_axis_name="c", subcore_axis_name="s")

    @functools.partial(
        pl.kernel,
        mesh=mesh,
        out_type=jax.ShapeDtypeStruct((n_rows, HID), jnp.float32),
        scratch_types=[
            pltpu.VMEM((chunks_per_w, _K), jnp.int32),
            pltpu.VMEM((_NB, _K, HID), jnp.float32),
        ] + [pltpu.SemaphoreType.DMA] * _NB,
    )
    def gather_kernel(idx_hbm, table_hbm, h_hbm, idx_v, rows_v, *gsems):
        wid = lax.axis_index("s") * _NC + lax.axis_index("c")
        chunk0 = wid * chunks_per_w
        row0 = wid * rows_per_w

        pltpu.sync_copy(idx_hbm.at[pl.ds(chunk0, chunks_per_w)], idx_v)

        for b in range(_NB):
            pltpu.async_copy(table_hbm.at[idx_v.at[b]], rows_v.at[b], gsems[b])

        def body(g, _):
            for b in range(_NB):
                j = g * _NB + b
                pltpu.make_async_copy(
                    table_hbm.at[idx_v.at[0]], rows_v.at[b], gsems[b]
                ).wait()
                pltpu.sync_copy(
                    rows_v.at[b], h_hbm.at[pl.ds(row0 + j * _K, _K)]
                )

                @pl.when(j + _NB < chunks_per_w)
                def _():
                    pltpu.async_copy(
                        table_hbm.at[idx_v.at[j + _NB]], rows_v.at[b], gsems[b]
                    )

            return 0

        lax.fori_loop(0, n_groups, body, 0)

    return gather_kernel(x2d, emb_table)


def _tc_project(h, W_up, b_up, blk):
    n = h.shape[0]

    def mm_kernel(h_ref, w_ref, b_ref, o_ref):
        hb = jnp.maximum(h_ref[...], 0.0)
        o_ref[...] = (
            jnp.dot(hb, w_ref[...], preferred_element_type=jnp.float32)
            + b_ref[...]
        )

    return pl.pallas_call(
        mm_kernel,
        grid=(n // blk,),
        in_specs=[
            pl.BlockSpec((blk, HID), lambda i: (i, 0)),
            pl.BlockSpec((HID, OUT), lambda i: (0, 0)),
            pl.BlockSpec((1, OUT), lambda i: (0, 0)),
        ],
        out_specs=pl.BlockSpec((blk, OUT), lambda i: (i, 0)),
        out_shape=jax.ShapeDtypeStruct((n, OUT), jnp.float32),
    )(h, W_up, b_up.reshape(1, OUT))


def kernel(x, emb_table, W_up, b_up):
    B, L = x.shape
    n = B * L
    x2d = x.reshape(n // _K, _K).astype(jnp.int32)
    h = _sc_gather(x2d, emb_table, n)
    out = _tc_project(h, W_up, b_up, blk=1024)
    return out.reshape(B, L, OUT)


# speedup vs baseline: 4.3279x; 1.2242x over previous
import functools

import jax
import jax.numpy as jnp
from jax import lax
from jax.experimental import pallas as pl
from jax.experimental.pallas import tpu as pltpu
from jax.experimental.pallas import tpu_sc as plsc

HID = 128
OUT = 512

_NC = 2
_NS = 16
_NW = _NC * _NS

_K = 128
_NB = 4


def _sc_gather(x2d, table, n_rows, dtype):
    n_chunks = x2d.shape[0]
    chunks_per_w = n_chunks // _NW
    rows_per_w = chunks_per_w * _K
    n_groups = chunks_per_w // _NB

    mesh = plsc.VectorSubcoreMesh(core_axis_name="c", subcore_axis_name="s")

    @functools.partial(
        pl.kernel,
        mesh=mesh,
        out_type=jax.ShapeDtypeStruct((n_rows, HID), dtype),
        scratch_types=[
            pltpu.VMEM((chunks_per_w, _K), jnp.int32),
            pltpu.VMEM((_NB, _K, HID), dtype),
        ] + [pltpu.SemaphoreType.DMA] * _NB,
    )
    def gather_kernel(idx_hbm, table_hbm, h_hbm, idx_v, rows_v, *gsems):
        wid = lax.axis_index("s") * _NC + lax.axis_index("c")
        chunk0 = wid * chunks_per_w
        row0 = wid * rows_per_w

        pltpu.sync_copy(idx_hbm.at[pl.ds(chunk0, chunks_per_w)], idx_v)

        for b in range(_NB):
            pltpu.async_copy(table_hbm.at[idx_v.at[b]], rows_v.at[b], gsems[b])

        def body(g, _):
            for b in range(_NB):
                j = g * _NB + b
                pltpu.make_async_copy(
                    table_hbm.at[idx_v.at[0]], rows_v.at[b], gsems[b]
                ).wait()
                pltpu.sync_copy(
                    rows_v.at[b], h_hbm.at[pl.ds(row0 + j * _K, _K)]
                )

                @pl.when(j + _NB < chunks_per_w)
                def _():
                    pltpu.async_copy(
                        table_hbm.at[idx_v.at[j + _NB]], rows_v.at[b], gsems[b]
                    )

            return 0

        lax.fori_loop(0, n_groups, body, 0)

    return gather_kernel(x2d, table)


def _tc_project(h_bf, W_up, b_up, blk):
    n = h_bf.shape[0]

    def mm_kernel(h_ref, w_ref, b_ref, o_ref):
        hb = jnp.maximum(h_ref[...], 0.0)
        o_ref[...] = (
            jnp.dot(hb, w_ref[...], preferred_element_type=jnp.float32)
            + b_ref[...]
        )

    return pl.pallas_call(
        mm_kernel,
        grid=(n // blk,),
        in_specs=[
            pl.BlockSpec((blk, HID), lambda i: (i, 0)),
            pl.BlockSpec((HID, OUT), lambda i: (0, 0)),
            pl.BlockSpec((1, OUT), lambda i: (0, 0)),
        ],
        out_specs=pl.BlockSpec((blk, OUT), lambda i: (i, 0)),
        out_shape=jax.ShapeDtypeStruct((n, OUT), jnp.float32),
    )(h_bf, W_up, b_up.reshape(1, OUT))


def kernel(x, emb_table, W_up, b_up):
    B, L = x.shape
    n = B * L
    x2d = x.reshape(n // _K, _K).astype(jnp.int32)
    h_bf = _sc_gather(x2d, emb_table, n, jnp.float32)
    out = _tc_project(h_bf, W_up, b_up, blk=2048)
    return out.reshape(B, L, OUT)


# speedup vs baseline: 4.6304x; 1.0699x over previous
import functools

import jax
import jax.numpy as jnp
from jax import lax
from jax.experimental import pallas as pl
from jax.experimental.pallas import tpu as pltpu
from jax.experimental.pallas import tpu_sc as plsc

HID = 128
OUT = 512

_NC = 2
_NS = 16
_NW = _NC * _NS

_K = 128
_NB = 4


def _sc_gather(x2d, table, n_rows, dtype):
    n_chunks = x2d.shape[0]
    chunks_per_w = n_chunks // _NW
    rows_per_w = chunks_per_w * _K
    n_groups = chunks_per_w // _NB

    mesh = plsc.VectorSubcoreMesh(core_axis_name="c", subcore_axis_name="s")

    @functools.partial(
        pl.kernel,
        mesh=mesh,
        out_type=jax.ShapeDtypeStruct((n_rows, HID), dtype),
        scratch_types=[
            pltpu.VMEM((chunks_per_w, _K), jnp.int32),
            pltpu.VMEM((_NB, _K, HID), dtype),
        ] + [pltpu.SemaphoreType.DMA] * _NB,
    )
    def gather_kernel(idx_hbm, table_hbm, h_hbm, idx_v, rows_v, *gsems):
        wid = lax.axis_index("s") * _NC + lax.axis_index("c")
        chunk0 = wid * chunks_per_w
        row0 = wid * rows_per_w

        pltpu.sync_copy(idx_hbm.at[pl.ds(chunk0, chunks_per_w)], idx_v)

        for b in range(_NB):
            pltpu.async_copy(table_hbm.at[idx_v.at[b]], rows_v.at[b], gsems[b])

        def body(g, _):
            for b in range(_NB):
                j = g * _NB + b
                pltpu.make_async_copy(
                    table_hbm.at[idx_v.at[0]], rows_v.at[b], gsems[b]
                ).wait()
                pltpu.sync_copy(
                    rows_v.at[b], h_hbm.at[pl.ds(row0 + j * _K, _K)]
                )

                @pl.when(j + _NB < chunks_per_w)
                def _():
                    pltpu.async_copy(
                        table_hbm.at[idx_v.at[j + _NB]], rows_v.at[b], gsems[b]
                    )

            return 0

        lax.fori_loop(0, n_groups, body, 0)

    return gather_kernel(x2d, table)


def _tc_project(h_bf, W_up, b_up, blk):
    n = h_bf.shape[0]

    def mm_kernel(h_ref, w_ref, b_ref, o_ref):
        hb = jnp.maximum(h_ref[...], 0.0)
        o_ref[...] = (
            jnp.dot(hb, w_ref[...], preferred_element_type=jnp.float32)
            + b_ref[...]
        )

    return pl.pallas_call(
        mm_kernel,
        grid=(n // blk,),
        in_specs=[
            pl.BlockSpec((blk, HID), lambda i: (i, 0)),
            pl.BlockSpec((HID, OUT), lambda i: (0, 0)),
            pl.BlockSpec((1, OUT), lambda i: (0, 0)),
        ],
        out_specs=pl.BlockSpec((blk, OUT), lambda i: (i, 0)),
        out_shape=jax.ShapeDtypeStruct((n, OUT), jnp.float32),
    )(h_bf, W_up, b_up.reshape(1, OUT))


def kernel(x, emb_table, W_up, b_up):
    B, L = x.shape
    n = B * L
    x2d = x.reshape(n // _K, _K).astype(jnp.int32)
    h_bf = _sc_gather(x2d, emb_table, n, jnp.float32)
    out = _tc_project(h_bf, W_up, b_up, blk=4096)
    return out.reshape(B, L, OUT)


# speedup vs baseline: 4.7274x; 1.0210x over previous
import functools

import jax
import jax.numpy as jnp
from jax import lax
from jax.experimental import pallas as pl
from jax.experimental.pallas import tpu as pltpu
from jax.experimental.pallas import tpu_sc as plsc

HID = 128
OUT = 512

_NC = 2
_NS = 16
_NW = _NC * _NS

_K = 128
_NB = 4


def _sc_gather(x2d, table, n_rows, dtype):
    n_chunks = x2d.shape[0]
    chunks_per_w = n_chunks // _NW
    rows_per_w = chunks_per_w * _K
    n_groups = chunks_per_w // _NB

    mesh = plsc.VectorSubcoreMesh(core_axis_name="c", subcore_axis_name="s")

    @functools.partial(
        pl.kernel,
        mesh=mesh,
        out_type=jax.ShapeDtypeStruct((n_rows, HID), dtype),
        scratch_types=[
            pltpu.VMEM((chunks_per_w, _K), jnp.int32),
            pltpu.VMEM((_NB, _K, HID), dtype),
        ] + [pltpu.SemaphoreType.DMA] * _NB,
    )
    def gather_kernel(idx_hbm, table_hbm, h_hbm, idx_v, rows_v, *gsems):
        wid = lax.axis_index("s") * _NC + lax.axis_index("c")
        chunk0 = wid * chunks_per_w
        row0 = wid * rows_per_w

        pltpu.sync_copy(idx_hbm.at[pl.ds(chunk0, chunks_per_w)], idx_v)

        for b in range(_NB):
            pltpu.async_copy(table_hbm.at[idx_v.at[b]], rows_v.at[b], gsems[b])

        def body(g, _):
            for b in range(_NB):
                j = g * _NB + b
                pltpu.make_async_copy(
                    table_hbm.at[idx_v.at[0]], rows_v.at[b], gsems[b]
                ).wait()
                pltpu.sync_copy(
                    rows_v.at[b], h_hbm.at[pl.ds(row0 + j * _K, _K)]
                )

                @pl.when(j + _NB < chunks_per_w)
                def _():
                    pltpu.async_copy(
                        table_hbm.at[idx_v.at[j + _NB]], rows_v.at[b], gsems[b]
                    )

            return 0

        lax.fori_loop(0, n_groups, body, 0)

    return gather_kernel(x2d, table)


def _tc_project(h_bf, W_up, b_up, blk):
    n = h_bf.shape[0]

    def mm_kernel(h_ref, w_ref, b_ref, o_ref):
        hb = jnp.maximum(h_ref[...], 0.0)
        o_ref[...] = (
            jnp.dot(hb, w_ref[...], preferred_element_type=jnp.float32)
            + b_ref[...]
        )

    return pl.pallas_call(
        mm_kernel,
        grid=(n // blk,),
        in_specs=[
            pl.BlockSpec((blk, HID), lambda i: (i, 0)),
            pl.BlockSpec((HID, OUT), lambda i: (0, 0)),
            pl.BlockSpec((1, OUT), lambda i: (0, 0)),
        ],
        out_specs=pl.BlockSpec((blk, OUT), lambda i: (i, 0)),
        out_shape=jax.ShapeDtypeStruct((n, OUT), jnp.float32),
    )(h_bf, W_up, b_up.reshape(1, OUT))


def kernel(x, emb_table, W_up, b_up):
    B, L = x.shape
    n = B * L
    x2d = x.reshape(n // _K, _K).astype(jnp.int32)
    h_bf = _sc_gather(x2d, emb_table, n, jnp.float32)
    out = _tc_project(h_bf, W_up, b_up, blk=8192)
    return out.reshape(B, L, OUT)
